# baseline (speedup 1.0000x reference)
import functools
import math

import jax
import jax.numpy as jnp
from jax.experimental import pallas as pl
from jax.experimental.pallas import tpu as pltpu

D_MODEL_K = 4096
MAX_NORM_K = math.sqrt(D_MODEL_K)
CHUNK = 1024
NBUF = 3
NSPLIT = 1


def _pipeline_kernel(ts_ref, x_hbm, tbl_hbm, o_hbm, buf, emb_ref,
                     in_sems, out_sems, row_sem, *, n_chunks, chunks_per_b,
                     n_batch):
    for b in range(n_batch):
        pltpu.make_async_copy(
            tbl_hbm.at[pl.ds(ts_ref[b], 1), :], emb_ref.at[pl.ds(b, 1), :],
            row_sem,
        ).start()

    sub = CHUNK // NSPLIT

    def copies_in(c, slot):
        return [
            pltpu.make_async_copy(
                x_hbm.at[pl.ds(c * CHUNK + k * sub, sub), :],
                buf.at[slot, pl.ds(k * sub, sub), :],
                in_sems.at[slot],
            )
            for k in range(NSPLIT)
        ]

    def copies_out(c, slot):
        return [
            pltpu.make_async_copy(
                buf.at[slot, pl.ds(k * sub, sub), :],
                o_hbm.at[pl.ds(c * CHUNK + k * sub, sub), :],
                out_sems.at[slot],
            )
            for k in range(NSPLIT)
        ]

    def start(cps):
        for cp in cps:
            cp.start()

    def wait(cps):
        for cp in cps:
            cp.wait()

    for s in range(min(NBUF, n_chunks)):
        start(copies_in(s, s))

    for b in range(n_batch):
        pltpu.make_async_copy(
            tbl_hbm.at[pl.ds(ts_ref[b], 1), :], emb_ref.at[pl.ds(b, 1), :],
            row_sem,
        ).wait()
    rows = emb_ref[...]
    norms = jnp.sqrt(jnp.sum(rows * rows, axis=-1, keepdims=True))
    emb_ref[...] = rows * jnp.where(norms > MAX_NORM_K,
                                    MAX_NORM_K / (norms + 1e-7), 1.0)

    plan = [(c * CHUNK, CHUNK) for c in range(n_chunks - 1)]
    plan += [((n_chunks - 1) * CHUNK, CHUNK // 2),
             ((n_chunks - 1) * CHUNK + CHUNK // 2, CHUNK // 2)]

    def pcopy_in(c, slot):
        row0, sz = plan[c]
        return pltpu.make_async_copy(
            x_hbm.at[pl.ds(row0, sz), :],
            buf.at[slot, pl.ds(0, sz), :], in_sems.at[slot])

    def pcopy_out(c, slot):
        row0, sz = plan[c]
        return pltpu.make_async_copy(
            buf.at[slot, pl.ds(0, sz), :],
            o_hbm.at[pl.ds(row0, sz), :], out_sems.at[slot])

    np_ = len(plan)
    for c in range(np_):
        slot = c % NBUF
        row0, sz = plan[c]
        b = row0 // (chunks_per_b * CHUNK)
        pcopy_in(c, slot).wait()
        buf[slot, pl.ds(0, sz), :] += emb_ref[pl.ds(b, 1), :]
        pcopy_out(c, slot).start()
        nxt = c + NBUF
        if nxt < np_:
            pcopy_out(c, slot).wait()
            pcopy_in(nxt, slot).start()

    for c in range(max(0, np_ - NBUF), np_):
        pcopy_out(c, c % NBUF).wait()


def kernel(x, timesteps, table):
    B, S, D = x.shape
    x2 = x.reshape(B * S, D)
    n_chunks = (B * S) // CHUNK
    chunks_per_b = S // CHUNK
    body = functools.partial(_pipeline_kernel, n_chunks=n_chunks,
                             chunks_per_b=chunks_per_b, n_batch=B)
    out = pl.pallas_call(
        body,
        grid_spec=pltpu.PrefetchScalarGridSpec(
            num_scalar_prefetch=1,
            grid=(1,),
            in_specs=[
                pl.BlockSpec(memory_space=pl.ANY),
                pl.BlockSpec(memory_space=pl.ANY),
            ],
            out_specs=pl.BlockSpec(memory_space=pl.ANY),
            scratch_shapes=[
                pltpu.VMEM((NBUF, CHUNK, D), x.dtype),
                pltpu.VMEM((B, D), x.dtype),
                pltpu.SemaphoreType.DMA((NBUF,)),
                pltpu.SemaphoreType.DMA((NBUF,)),
                pltpu.SemaphoreType.DMA,
            ],
        ),
        out_shape=jax.ShapeDtypeStruct(x2.shape, x.dtype),
        compiler_params=pltpu.CompilerParams(
            vmem_limit_bytes=128 * 1024 * 1024,
        ),
    )(timesteps, x2, table)
    return out.reshape(B, S, D)
